# baseline (speedup 1.0000x reference)
import functools

import jax
import jax.numpy as jnp
from jax import lax
from jax.experimental import pallas as pl
from jax.experimental.pallas import tpu as pltpu
from jax.experimental.pallas import tpu_sc as plsc

N_NODES = 10000
N_PAD = 10240
E_TOTAL = 160000
CH = 256
NC, NS = 2, 16
LANES = 16

EPT = E_TOTAL // (NC * NS)
DEG_ITERS = EPT // LANES + 1
SLICE = N_PAD // NS

RPT = N_PAD // (NC * NS)
CHUNK = 32
CHUNK_LOG = 5
RING = 2048
RING_ROWS = RING // CHUNK
EB = 3200
NBLK = E_TOTAL // EB
SB = 800
SBVR = SB // LANES
CU = 5

def _deg_body(dst_hbm, deg_hbm, dstv, degloc, tmpv, accv, deg_sh):
    c = lax.axis_index("c")
    s = lax.axis_index("s")
    wid = c * NS + s
    lane = lax.iota(jnp.int32, LANES)
    zeros = jnp.zeros((LANES,), jnp.float32)
    ones = jnp.ones((LANES,), jnp.float32)

    def _zero(i, _):
        degloc[pl.ds(i * LANES, LANES)] = zeros
        return 0
    lax.fori_loop(0, N_PAD // LANES, _zero, 0)

    pltpu.sync_copy(dst_hbm.at[pl.ds(wid * EPT, EPT)], dstv.at[pl.ds(0, EPT)])

    def _count(i, _):
        idx = dstv[pl.ds(i * LANES, LANES)]
        m = (i * LANES + lane) < EPT
        idx = jnp.where(m, idx, 0)
        plsc.addupdate_scatter(degloc, [idx], ones, mask=m)
        return 0
    lax.fori_loop(0, DEG_ITERS, _count, 0)

    pltpu.sync_copy(degloc, deg_sh.at[pl.ds(s * N_PAD, N_PAD)])
    plsc.subcore_barrier()

    base = s * SLICE

    def _zacc(i, _):
        accv[pl.ds(i * LANES, LANES)] = zeros
        return 0
    lax.fori_loop(0, SLICE // LANES, _zacc, 0)

    def _red(j, _):
        pltpu.sync_copy(deg_sh.at[pl.ds(j * N_PAD + base, SLICE)], tmpv)

        def _add(i, _):
            accv[pl.ds(i * LANES, LANES)] = (
                accv[pl.ds(i * LANES, LANES)] + tmpv[pl.ds(i * LANES, LANES)])
            return 0
        lax.fori_loop(0, SLICE // LANES, _add, 0)
        return 0
    lax.fori_loop(0, NS, _red, 0)

    pltpu.sync_copy(accv, deg_hbm.at[pl.ds(c * N_PAD + base, SLICE)])


def _scatter_body(y_hbm, src_hbm, dst_hbm, acc_hbm,
                  srcv, dstv, csrc, cdst, rows, acc, semg, semes, semed):
    c = lax.axis_index("c")
    s = lax.axis_index("s")
    wid = c * NS + s
    base = wid * RPT

    pltpu.sync_copy(y_hbm.at[pl.ds(base, RPT)], acc.at[pl.ds(0, RPT)])

    def _stage_start(blk):
        p = blk & 1
        e0 = blk * EB
        pltpu.async_copy(src_hbm.at[pl.ds(e0, EB)],
                         srcv.at[pl.ds(p * EB, EB)], semes.at[p])
        pltpu.async_copy(dst_hbm.at[pl.ds(e0, EB)],
                         dstv.at[pl.ds(p * EB, EB)], semed.at[p])

    def _stage_wait(blk):
        p = blk & 1
        e0 = blk * EB
        pltpu.make_async_copy(
            src_hbm.at[pl.ds(e0, EB)],
            srcv.at[pl.ds(p * EB, EB)], semes.at[p]).wait()
        pltpu.make_async_copy(
            dst_hbm.at[pl.ds(e0, EB)],
            dstv.at[pl.ds(p * EB, EB)], semed.at[p]).wait()

    def _fire(k):
        p = k & 1
        r = k & (RING_ROWS - 1)
        pltpu.async_copy(y_hbm.at[csrc.at[r]],
                         rows.at[pl.ds(p * CHUNK, CHUNK)], semg.at[p])

    def _accw(k):
        p = k & 1
        r = k & (RING_ROWS - 1)
        pltpu.make_async_copy(
            y_hbm.at[csrc.at[r]],
            rows.at[pl.ds(p * CHUNK, CHUNK)], semg.at[p]).wait()

        def _grp(g, _):
            dlv = cdst[r, pl.ds(g * LANES, LANES)]
            dls = [dlv[q] for q in range(LANES)]
            nv = CH // LANES
            for q in range(LANES):
                e = g * LANES + q
                vals = {}
                for v in range(nv + 2):
                    if v < nv:
                        vals[v] = rows[p * CHUNK + e, pl.ds(v * LANES, LANES)]
                    if v >= 2:
                        plsc.addupdate(
                            acc.at[dls[q], pl.ds((v - 2) * LANES, LANES)],
                            vals[v - 2])
            return 0
        lax.fori_loop(0, CHUNK // LANES, _grp, 0)

    def _compact(p):
        def body(i, cntv):
            ds_ = []
            svs = []
            ms = []
            for u in range(CU):
                d = dstv[pl.ds(p * EB + (i * CU + u) * LANES, LANES)]
                sv = srcv[pl.ds(p * EB + (i * CU + u) * LANES, LANES)]
                ds_.append(d)
                svs.append(sv)
                ms.append((d >= base) & (d < base + RPT))
            cums = [plsc.cumsum(m.astype(jnp.int32)) for m in ms]
            pops = [plsc.all_reduce_population_count(m) for m in ms]
            for u in range(CU):
                pos = cntv + cums[u] - ms[u].astype(jnp.int32)
                rr = (pos >> CHUNK_LOG) & (RING_ROWS - 1)
                cc = pos & (CHUNK - 1)
                plsc.store_scatter(csrc, [rr, cc], svs[u], mask=ms[u])
                plsc.store_scatter(cdst, [rr, cc], ds_[u] - base, mask=ms[u])
                cntv = cntv + pops[u]
            return cntv
        return body

    def _pump(f, avail):
        def _step(f2):
            _fire(f2)

            @pl.when(f2 >= 1)
            def _():
                _accw(f2 - 1)
            return f2 + 1
        return lax.while_loop(lambda f2: f2 < avail, _step, f)

    _stage_start(0)

    def _block(blk, carry):
        cnt, fired = carry
        p = blk & 1
        _stage_wait(blk)

        @pl.when(blk + 1 < NBLK)
        def _():
            _stage_start(blk + 1)

        def _sub(sb, carry2):
            cnt2, fired2 = carry2
            cntv = lax.fori_loop(sb * (SBVR // CU), (sb + 1) * (SBVR // CU),
                                 _compact(p),
                                 jnp.full((LANES,), cnt2, jnp.int32))
            cnt2 = jnp.max(cntv)
            fired2 = _pump(fired2, cnt2 >> CHUNK_LOG)
            return (cnt2, fired2)
        return lax.fori_loop(0, EB // SB, _sub, (cnt, fired))

    cnt, fired = lax.fori_loop(0, NBLK, _block,
                               (jnp.int32(0), jnp.int32(0)))

    @pl.when(fired >= 1)
    def _():
        _accw(fired - 1)

    lane = lax.iota(jnp.int32, LANES)
    cnt_pad = ((cnt + CHUNK - 1) >> CHUNK_LOG) << CHUNK_LOG

    @pl.when(cnt_pad > fired * CHUNK)
    def _():
        for j in range(CHUNK // LANES):
            posv = cnt + j * LANES + lane
            mf = posv < cnt_pad
            rr = (posv >> CHUNK_LOG) & (RING_ROWS - 1)
            cc = posv & (CHUNK - 1)
            plsc.store_scatter(csrc, [rr, cc],
                               jnp.zeros((LANES,), jnp.int32), mask=mf)
            plsc.store_scatter(cdst, [rr, cc],
                               jnp.full((LANES,), RPT, jnp.int32), mask=mf)
        _fire(fired)
        _accw(fired)

    pltpu.sync_copy(acc.at[pl.ds(0, RPT)], acc_hbm.at[pl.ds(base, RPT)])


BLK = 1024


def _mm_body(x_ref, w_ref, d2_ref, y_ref):
    d = d2_ref[:, 0:1] + d2_ref[:, 1:2] + 1.0
    disq = lax.rsqrt(d)
    y_ref[...] = jnp.dot(x_ref[...], w_ref[...],
                         preferred_element_type=jnp.float32) * disq


_mm = pl.pallas_call(
    _mm_body,
    grid=(N_PAD // BLK,),
    in_specs=[
        pl.BlockSpec((BLK, CH), lambda i: (i, 0)),
        pl.BlockSpec((CH, CH), lambda i: (0, 0)),
        pl.BlockSpec((BLK, 2), lambda i: (i, 0)),
    ],
    out_specs=pl.BlockSpec((BLK, CH), lambda i: (i, 0)),
    out_shape=jax.ShapeDtypeStruct((N_PAD, CH), jnp.float32),
)


def _fin_body(a_ref, d2_ref, b_ref, o_ref):
    d = d2_ref[:, 0:1] + d2_ref[:, 1:2] + 1.0
    disq = lax.rsqrt(d)
    o_ref[...] = jnp.maximum(a_ref[...] * disq + b_ref[...], 0.0)


_fin = pl.pallas_call(
    _fin_body,
    grid=(N_PAD // BLK,),
    in_specs=[
        pl.BlockSpec((BLK, CH), lambda i: (i, 0)),
        pl.BlockSpec((BLK, 2), lambda i: (i, 0)),
        pl.BlockSpec((1, CH), lambda i: (0, 0)),
    ],
    out_specs=pl.BlockSpec((BLK, CH), lambda i: (i, 0)),
    out_shape=jax.ShapeDtypeStruct((N_PAD, CH), jnp.float32),
)


@functools.lru_cache(maxsize=1)
def _build_sc_kernels():
    mesh = plsc.VectorSubcoreMesh(
        core_axis_name="c", subcore_axis_name="s",
        num_cores=NC, num_subcores=NS)
    deg_kernel = pl.kernel(
        _deg_body,
        out_type=jax.ShapeDtypeStruct((NC * N_PAD,), jnp.float32),
        mesh=mesh,
        compiler_params=pltpu.CompilerParams(needs_layout_passes=False),
        scratch_types=[
            pltpu.VMEM((EPT + LANES,), jnp.int32),
            pltpu.VMEM((N_PAD,), jnp.float32),
            pltpu.VMEM((SLICE,), jnp.float32),
            pltpu.VMEM((SLICE,), jnp.float32),
            pltpu.VMEM_SHARED((NS * N_PAD,), jnp.float32),
        ],
    )
    scatter_kernel = pl.kernel(
        _scatter_body,
        out_type=jax.ShapeDtypeStruct((N_PAD, CH), jnp.float32),
        mesh=mesh,
        compiler_params=pltpu.CompilerParams(needs_layout_passes=False),
        scratch_types=[
            pltpu.VMEM((2 * EB,), jnp.int32),
            pltpu.VMEM((2 * EB,), jnp.int32),
            pltpu.VMEM((RING_ROWS, CHUNK), jnp.int32),
            pltpu.VMEM((RING_ROWS, CHUNK), jnp.int32),
            pltpu.VMEM((2 * CHUNK, CH), jnp.float32),
            pltpu.VMEM((RPT + 1, CH), jnp.float32),
            pltpu.SemaphoreType.DMA((2,)),
            pltpu.SemaphoreType.DMA((2,)),
            pltpu.SemaphoreType.DMA((2,)),
        ],
    )
    return deg_kernel, scatter_kernel


def kernel(x, edge_index, W, b):
    n = x.shape[0]
    src = edge_index[0].astype(jnp.int32)
    dst = edge_index[1].astype(jnp.int32)
    xp = jnp.pad(x, ((0, N_PAD - n), (0, 0)))

    _deg_kernel, _scatter_kernel = _build_sc_kernels()
    degflat = _deg_kernel(dst)
    d2t = degflat.reshape(NC, N_PAD).T

    y = _mm(xp, W, d2t)
    acc = _scatter_kernel(y, src, dst)
    out = _fin(acc, d2t, b.reshape(1, CH))
    return out[:n]


# speedup vs baseline: 1.0803x; 1.0803x over previous
import functools

import jax
import jax.numpy as jnp
from jax import lax
from jax.experimental import pallas as pl
from jax.experimental.pallas import tpu as pltpu
from jax.experimental.pallas import tpu_sc as plsc

N_NODES = 10000
N_PAD = 10240
E_TOTAL = 160000
CH = 256
NC, NS = 2, 16
LANES = 16

EPT = E_TOTAL // (NC * NS)
DEG_ITERS = EPT // LANES + 1
SLICE = N_PAD // NS

RPT = N_PAD // (NC * NS)
CHUNK = 32
CHUNK_LOG = 5
RING = 2048
RING_ROWS = RING // CHUNK
EB = 3200
NBLK = E_TOTAL // EB
SB = 800
SBVR = SB // LANES
CU = 5
LRPT = N_NODES - (NC * NS - 1) * RPT

def _deg_body(dst_hbm, deg_hbm, dstv, degloc, tmpv, accv, deg_sh):
    c = lax.axis_index("c")
    s = lax.axis_index("s")
    wid = c * NS + s
    lane = lax.iota(jnp.int32, LANES)
    zeros = jnp.zeros((LANES,), jnp.float32)
    ones = jnp.ones((LANES,), jnp.float32)

    def _zero(i, _):
        degloc[pl.ds(i * LANES, LANES)] = zeros
        return 0
    lax.fori_loop(0, N_PAD // LANES, _zero, 0)

    pltpu.sync_copy(dst_hbm.at[pl.ds(wid * EPT, EPT)], dstv.at[pl.ds(0, EPT)])

    def _count(i, _):
        idx = dstv[pl.ds(i * LANES, LANES)]
        m = (i * LANES + lane) < EPT
        idx = jnp.where(m, idx, 0)
        plsc.addupdate_scatter(degloc, [idx], ones, mask=m)
        return 0
    lax.fori_loop(0, DEG_ITERS, _count, 0)

    pltpu.sync_copy(degloc, deg_sh.at[pl.ds(s * N_PAD, N_PAD)])
    plsc.subcore_barrier()

    base = s * SLICE

    def _zacc(i, _):
        accv[pl.ds(i * LANES, LANES)] = zeros
        return 0
    lax.fori_loop(0, SLICE // LANES, _zacc, 0)

    def _red(j, _):
        pltpu.sync_copy(deg_sh.at[pl.ds(j * N_PAD + base, SLICE)], tmpv)

        def _add(i, _):
            accv[pl.ds(i * LANES, LANES)] = (
                accv[pl.ds(i * LANES, LANES)] + tmpv[pl.ds(i * LANES, LANES)])
            return 0
        lax.fori_loop(0, SLICE // LANES, _add, 0)
        return 0
    lax.fori_loop(0, NS, _red, 0)

    pltpu.sync_copy(accv, deg_hbm.at[pl.ds(c * N_PAD + base, SLICE)])


def _scatter_body(y_hbm, src_hbm, dst_hbm, acc_hbm,
                  srcv, dstv, csrc, cdst, rows, acc, semg, semes, semed):
    c = lax.axis_index("c")
    s = lax.axis_index("s")
    wid = c * NS + s
    base = wid * RPT

    @pl.when(wid < NC * NS - 1)
    def _():
        pltpu.sync_copy(y_hbm.at[pl.ds(base, RPT)], acc.at[pl.ds(0, RPT)])

    @pl.when(wid == NC * NS - 1)
    def _():
        pltpu.sync_copy(y_hbm.at[pl.ds(base, LRPT)], acc.at[pl.ds(0, LRPT)])

    def _stage_start(blk):
        p = blk & 1
        e0 = blk * EB
        pltpu.async_copy(src_hbm.at[pl.ds(e0, EB)],
                         srcv.at[pl.ds(p * EB, EB)], semes.at[p])
        pltpu.async_copy(dst_hbm.at[pl.ds(e0, EB)],
                         dstv.at[pl.ds(p * EB, EB)], semed.at[p])

    def _stage_wait(blk):
        p = blk & 1
        e0 = blk * EB
        pltpu.make_async_copy(
            src_hbm.at[pl.ds(e0, EB)],
            srcv.at[pl.ds(p * EB, EB)], semes.at[p]).wait()
        pltpu.make_async_copy(
            dst_hbm.at[pl.ds(e0, EB)],
            dstv.at[pl.ds(p * EB, EB)], semed.at[p]).wait()

    def _fire(k):
        p = k & 1
        r = k & (RING_ROWS - 1)
        pltpu.async_copy(y_hbm.at[csrc.at[r]],
                         rows.at[pl.ds(p * CHUNK, CHUNK)], semg.at[p])

    def _accw(k):
        p = k & 1
        r = k & (RING_ROWS - 1)
        pltpu.make_async_copy(
            y_hbm.at[csrc.at[r]],
            rows.at[pl.ds(p * CHUNK, CHUNK)], semg.at[p]).wait()

        def _grp(g, _):
            dlv = cdst[r, pl.ds(g * LANES, LANES)]
            dls = [dlv[q] for q in range(LANES)]
            for q in range(LANES):
                e = g * LANES + q
                vals = [rows[p * CHUNK + e, pl.ds(v * LANES, LANES)]
                        for v in range(CH // LANES)]
                for v in range(CH // LANES):
                    plsc.addupdate(acc.at[dls[q], pl.ds(v * LANES, LANES)],
                                   vals[v])
            return 0
        lax.fori_loop(0, CHUNK // LANES, _grp, 0)

    def _compact(p):
        def body(i, cntv):
            ds_ = []
            svs = []
            ms = []
            for u in range(CU):
                d = dstv[pl.ds(p * EB + (i * CU + u) * LANES, LANES)]
                sv = srcv[pl.ds(p * EB + (i * CU + u) * LANES, LANES)]
                ds_.append(d)
                svs.append(sv)
                ms.append((d >= base) & (d < base + RPT))
            cums = [plsc.cumsum(m.astype(jnp.int32)) for m in ms]
            pops = [plsc.all_reduce_population_count(m) for m in ms]
            for u in range(CU):
                pos = cntv + cums[u] - ms[u].astype(jnp.int32)
                rr = (pos >> CHUNK_LOG) & (RING_ROWS - 1)
                cc = pos & (CHUNK - 1)
                plsc.store_scatter(csrc, [rr, cc], svs[u], mask=ms[u])
                plsc.store_scatter(cdst, [rr, cc], ds_[u] - base, mask=ms[u])
                cntv = cntv + pops[u]
            return cntv
        return body

    def _pump(f, avail):
        def _step(f2):
            _fire(f2)

            @pl.when(f2 >= 1)
            def _():
                _accw(f2 - 1)
            return f2 + 1
        return lax.while_loop(lambda f2: f2 < avail, _step, f)

    _stage_start(0)

    def _block(blk, carry):
        cnt, fired = carry
        p = blk & 1
        _stage_wait(blk)

        @pl.when(blk + 1 < NBLK)
        def _():
            _stage_start(blk + 1)

        def _sub(sb, carry2):
            cnt2, fired2 = carry2
            cntv = lax.fori_loop(sb * (SBVR // CU), (sb + 1) * (SBVR // CU),
                                 _compact(p),
                                 jnp.full((LANES,), cnt2, jnp.int32))
            cnt2 = jnp.max(cntv)
            fired2 = _pump(fired2, cnt2 >> CHUNK_LOG)
            return (cnt2, fired2)
        return lax.fori_loop(0, EB // SB, _sub, (cnt, fired))

    cnt, fired = lax.fori_loop(0, NBLK, _block,
                               (jnp.int32(0), jnp.int32(0)))

    @pl.when(fired >= 1)
    def _():
        _accw(fired - 1)

    lane = lax.iota(jnp.int32, LANES)
    cnt_pad = ((cnt + CHUNK - 1) >> CHUNK_LOG) << CHUNK_LOG

    @pl.when(cnt_pad > fired * CHUNK)
    def _():
        for j in range(CHUNK // LANES):
            posv = cnt + j * LANES + lane
            mf = posv < cnt_pad
            rr = (posv >> CHUNK_LOG) & (RING_ROWS - 1)
            cc = posv & (CHUNK - 1)
            plsc.store_scatter(csrc, [rr, cc],
                               jnp.zeros((LANES,), jnp.int32), mask=mf)
            plsc.store_scatter(cdst, [rr, cc],
                               jnp.full((LANES,), RPT, jnp.int32), mask=mf)
        _fire(fired)
        _accw(fired)

    @pl.when(wid < NC * NS - 1)
    def _():
        pltpu.sync_copy(acc.at[pl.ds(0, RPT)], acc_hbm.at[pl.ds(base, RPT)])

    @pl.when(wid == NC * NS - 1)
    def _():
        pltpu.sync_copy(acc.at[pl.ds(0, LRPT)], acc_hbm.at[pl.ds(base, LRPT)])


BLK = 2000


def _mm_body(x_ref, w_ref, d2_ref, y_ref):
    d = d2_ref[:, 0:1] + d2_ref[:, 1:2] + 1.0
    disq = lax.rsqrt(d)
    y_ref[...] = jnp.dot(x_ref[...], w_ref[...],
                         preferred_element_type=jnp.float32) * disq


_mm = pl.pallas_call(
    _mm_body,
    grid=(N_NODES // BLK,),
    in_specs=[
        pl.BlockSpec((BLK, CH), lambda i: (i, 0)),
        pl.BlockSpec((CH, CH), lambda i: (0, 0)),
        pl.BlockSpec((BLK, 2), lambda i: (i, 0)),
    ],
    out_specs=pl.BlockSpec((BLK, CH), lambda i: (i, 0)),
    out_shape=jax.ShapeDtypeStruct((N_NODES, CH), jnp.float32),
)


def _fin_body(a_ref, d2_ref, b_ref, o_ref):
    d = d2_ref[:, 0:1] + d2_ref[:, 1:2] + 1.0
    disq = lax.rsqrt(d)
    o_ref[...] = jnp.maximum(a_ref[...] * disq + b_ref[...], 0.0)


_fin = pl.pallas_call(
    _fin_body,
    grid=(N_NODES // BLK,),
    in_specs=[
        pl.BlockSpec((BLK, CH), lambda i: (i, 0)),
        pl.BlockSpec((BLK, 2), lambda i: (i, 0)),
        pl.BlockSpec((1, CH), lambda i: (0, 0)),
    ],
    out_specs=pl.BlockSpec((BLK, CH), lambda i: (i, 0)),
    out_shape=jax.ShapeDtypeStruct((N_NODES, CH), jnp.float32),
)


@functools.lru_cache(maxsize=1)
def _build_sc_kernels():
    mesh = plsc.VectorSubcoreMesh(
        core_axis_name="c", subcore_axis_name="s",
        num_cores=NC, num_subcores=NS)
    deg_kernel = pl.kernel(
        _deg_body,
        out_type=jax.ShapeDtypeStruct((NC * N_PAD,), jnp.float32),
        mesh=mesh,
        compiler_params=pltpu.CompilerParams(needs_layout_passes=False),
        scratch_types=[
            pltpu.VMEM((EPT + LANES,), jnp.int32),
            pltpu.VMEM((N_PAD,), jnp.float32),
            pltpu.VMEM((SLICE,), jnp.float32),
            pltpu.VMEM((SLICE,), jnp.float32),
            pltpu.VMEM_SHARED((NS * N_PAD,), jnp.float32),
        ],
    )
    scatter_kernel = pl.kernel(
        _scatter_body,
        out_type=jax.ShapeDtypeStruct((N_NODES, CH), jnp.float32),
        mesh=mesh,
        compiler_params=pltpu.CompilerParams(needs_layout_passes=False),
        scratch_types=[
            pltpu.VMEM((2 * EB,), jnp.int32),
            pltpu.VMEM((2 * EB,), jnp.int32),
            pltpu.VMEM((RING_ROWS, CHUNK), jnp.int32),
            pltpu.VMEM((RING_ROWS, CHUNK), jnp.int32),
            pltpu.VMEM((2 * CHUNK, CH), jnp.float32),
            pltpu.VMEM((RPT + 1, CH), jnp.float32),
            pltpu.SemaphoreType.DMA((2,)),
            pltpu.SemaphoreType.DMA((2,)),
            pltpu.SemaphoreType.DMA((2,)),
        ],
    )
    return deg_kernel, scatter_kernel


def kernel(x, edge_index, W, b):
    src = edge_index[0].astype(jnp.int32)
    dst = edge_index[1].astype(jnp.int32)

    _deg_kernel, _scatter_kernel = _build_sc_kernels()
    degflat = _deg_kernel(dst)
    d2t = degflat.reshape(NC, N_PAD).T

    y = _mm(x, W, d2t)
    acc = _scatter_kernel(y, src, dst)
    return _fin(acc, d2t, b.reshape(1, CH))
